# Initial kernel scaffold; baseline (speedup 1.0000x reference)
import jax
import jax.numpy as jnp
from jax.experimental import pallas as pl


def kernel(x, coordinates_cat):
    raise NotImplementedError("write your pallas kernel here")


# speedup vs baseline: 7.1215x; 7.1215x over previous
import jax
import jax.numpy as jnp
from jax.experimental import pallas as pl
from jax.experimental.pallas import tpu as pltpu

BATCH = 4
CHANNELS = 256
HH = 56
WW = 56
RATS = ((8, 8), (12, 12), (16, 16))
NSEL = (2, 3, 2)
OUTS = tuple(HH - h + 1 for (h, _) in RATS)
GOFF = (0, 2401, 4426)
CB = 32
NCB = CHANNELS // CB


def _nms_group(m, h, w, out_w, goff, nsel):
    ri = jax.lax.broadcasted_iota(jnp.int32, (out_w, out_w), 0)
    ci = jax.lax.broadcasted_iota(jnp.int32, (out_w, out_w), 1)
    lin = ri * out_w + ci
    sels, scs = [], []
    for _ in range(nsel):
        mx = jnp.max(m)
        sel = jnp.max(jnp.where(m == mx, lin, jnp.int32(-1)))
        si = sel // out_w
        sj = sel - si * out_w
        dl1 = jnp.maximum(h - jnp.abs(ri - si), 0)
        dl2 = jnp.maximum(w - jnp.abs(ci - sj), 0)
        kill = (5 * dl1 * dl2 > 2 * h * w) | (lin == sel)
        m = jnp.where(kill, -jnp.inf, m)
        sels.append(sel + goff)
        scs.append(mx)
    return sels, scs


def _body(x_ref, m8_ref, m12_ref, m16_ref, idx_ref, sc_ref, smap_ref):
    c = pl.program_id(1)
    part = jnp.sum(x_ref[0], axis=0)

    @pl.when(c == 0)
    def _():
        smap_ref[...] = part

    @pl.when(c > 0)
    def _():
        smap_ref[...] = smap_ref[...] + part

    @pl.when(c == NCB - 1)
    def _():
        smap = smap_ref[...]
        maps = []
        for (h, w), out_w, mref in zip(RATS, OUTS, (m8_ref, m12_ref, m16_ref)):
            acc = smap[:, 0:out_w]
            for dj in range(1, w):
                acc = acc + smap[:, dj:dj + out_w]
            accv = acc[0:out_w, :]
            for di in range(1, h):
                accv = accv + acc[di:di + out_w, :]
            pooled = accv / jnp.float32(h * w)
            mref[0] = pooled
            maps.append(pooled)

        all_sels, all_scs = [], []
        for g in range(3):
            h, w = RATS[g]
            sels, scs = _nms_group(maps[g], h, w, OUTS[g], GOFF[g], NSEL[g])
            all_sels += sels
            all_scs += scs
        all_sels.append(jnp.int32(0))
        all_scs.append(jnp.float32(0))
        for t in range(8):
            idx_ref[0, 0, t] = all_sels[t]
            sc_ref[0, 0, t] = all_scs[t]


def _run(x):
    grid = (BATCH, NCB)
    return pl.pallas_call(
        _body,
        grid=grid,
        in_specs=[pl.BlockSpec((1, CB, HH, WW), lambda b, c: (b, c, 0, 0))],
        out_specs=[
            pl.BlockSpec((1, OUTS[0], OUTS[0]), lambda b, c: (b, 0, 0)),
            pl.BlockSpec((1, OUTS[1], OUTS[1]), lambda b, c: (b, 0, 0)),
            pl.BlockSpec((1, OUTS[2], OUTS[2]), lambda b, c: (b, 0, 0)),
            pl.BlockSpec((1, 1, 8), lambda b, c: (b, 0, 0), memory_space=pltpu.SMEM),
            pl.BlockSpec((1, 1, 8), lambda b, c: (b, 0, 0), memory_space=pltpu.SMEM),
        ],
        out_shape=[
            jax.ShapeDtypeStruct((BATCH, OUTS[0], OUTS[0]), jnp.float32),
            jax.ShapeDtypeStruct((BATCH, OUTS[1], OUTS[1]), jnp.float32),
            jax.ShapeDtypeStruct((BATCH, OUTS[2], OUTS[2]), jnp.float32),
            jax.ShapeDtypeStruct((BATCH, 1, 8), jnp.int32),
            jax.ShapeDtypeStruct((BATCH, 1, 8), jnp.float32),
        ],
        scratch_shapes=[pltpu.VMEM((HH, WW), jnp.float32)],
        compiler_params=pltpu.CompilerParams(
            dimension_semantics=("parallel", "arbitrary")),
    )(x)


def kernel(x, coordinates_cat):
    m8, m12, m16, idx, sc = _run(x)
    window_scores = jnp.concatenate(
        [m8.reshape(BATCH, -1), m12.reshape(BATCH, -1), m16.reshape(BATCH, -1)],
        axis=1)
    return (idx[:, 0, :7], sc[:, 0, :7], window_scores)


# speedup vs baseline: 9.9243x; 1.3936x over previous
import jax
import jax.numpy as jnp
from jax.experimental import pallas as pl
from jax.experimental.pallas import tpu as pltpu

BATCH = 4
CHANNELS = 256
HH = 56
WW = 56
RATS = ((8, 8), (12, 12), (16, 16))
NSEL = (2, 3, 2)
OUTS = tuple(HH - h + 1 for (h, _) in RATS)
GOFF = (0, 2401, 4426)
CB = 32
NCB = CHANNELS // CB


def _nms_group(m, h, w, out_w, goff, nsel):
    ri = jax.lax.broadcasted_iota(jnp.int32, (out_w, out_w), 0)
    ci = jax.lax.broadcasted_iota(jnp.int32, (out_w, out_w), 1)
    lin = ri * out_w + ci
    sels, scs = [], []
    for _ in range(nsel):
        mx = jnp.max(m)
        sel = jnp.max(jnp.where(m == mx, lin, jnp.int32(-1)))
        si = sel // out_w
        sj = sel - si * out_w
        dl1 = jnp.maximum(h - jnp.abs(ri - si), 0)
        dl2 = jnp.maximum(w - jnp.abs(ci - sj), 0)
        kill = (5 * dl1 * dl2 > 2 * h * w) | (lin == sel)
        m = jnp.where(kill, -jnp.inf, m)
        sels.append(sel + goff)
        scs.append(mx)
    return sels, scs


def _body(x_ref, m8_ref, m12_ref, m16_ref, idx_ref, sc_ref):
    if True:
        smap = jnp.sum(x_ref[0], axis=0)
        maps = []
        for (h, w), out_w, mref in zip(RATS, OUTS, (m8_ref, m12_ref, m16_ref)):
            acc = smap[:, 0:out_w]
            for dj in range(1, w):
                acc = acc + smap[:, dj:dj + out_w]
            accv = acc[0:out_w, :]
            for di in range(1, h):
                accv = accv + acc[di:di + out_w, :]
            pooled = accv / jnp.float32(h * w)
            mref[0] = pooled
            maps.append(pooled)

        all_sels, all_scs = [], []
        for g in range(3):
            h, w = RATS[g]
            sels, scs = _nms_group(maps[g], h, w, OUTS[g], GOFF[g], NSEL[g])
            all_sels += sels
            all_scs += scs
        all_sels.append(jnp.int32(0))
        all_scs.append(jnp.float32(0))
        for t in range(8):
            idx_ref[0, 0, t] = all_sels[t]
            sc_ref[0, 0, t] = all_scs[t]


def _run(x):
    grid = (BATCH,)
    return pl.pallas_call(
        _body,
        grid=grid,
        in_specs=[pl.BlockSpec((1, CHANNELS, HH, WW), lambda b: (b, 0, 0, 0))],
        out_specs=[
            pl.BlockSpec((1, OUTS[0], OUTS[0]), lambda b: (b, 0, 0)),
            pl.BlockSpec((1, OUTS[1], OUTS[1]), lambda b: (b, 0, 0)),
            pl.BlockSpec((1, OUTS[2], OUTS[2]), lambda b: (b, 0, 0)),
            pl.BlockSpec((1, 1, 8), lambda b: (b, 0, 0), memory_space=pltpu.SMEM),
            pl.BlockSpec((1, 1, 8), lambda b: (b, 0, 0), memory_space=pltpu.SMEM),
        ],
        out_shape=[
            jax.ShapeDtypeStruct((BATCH, OUTS[0], OUTS[0]), jnp.float32),
            jax.ShapeDtypeStruct((BATCH, OUTS[1], OUTS[1]), jnp.float32),
            jax.ShapeDtypeStruct((BATCH, OUTS[2], OUTS[2]), jnp.float32),
            jax.ShapeDtypeStruct((BATCH, 1, 8), jnp.int32),
            jax.ShapeDtypeStruct((BATCH, 1, 8), jnp.float32),
        ],
        compiler_params=pltpu.CompilerParams(
            dimension_semantics=("parallel",)),
    )(x)


def kernel(x, coordinates_cat):
    m8, m12, m16, idx, sc = _run(x)
    window_scores = jnp.concatenate(
        [m8.reshape(BATCH, -1), m12.reshape(BATCH, -1), m16.reshape(BATCH, -1)],
        axis=1)
    return (idx[:, 0, :7], sc[:, 0, :7], window_scores)
